# Initial kernel scaffold; baseline (speedup 1.0000x reference)
import jax
import jax.numpy as jnp
from jax.experimental import pallas as pl


def kernel(cat_data, cont_data, tables, W1, b1, g1, be1, W2, b2, g2, be2, W3, b3, g3, be3, W4, b4, W5, b5):
    raise NotImplementedError("write your pallas kernel here")


# speedup vs baseline: 3.9103x; 3.9103x over previous
import functools

import jax
import jax.numpy as jnp
from jax import lax
from jax.experimental import pallas as pl
from jax.experimental.pallas import tpu as pltpu
from jax.experimental.pallas import tpu_sc as plsc

B = 16384
N_FIELDS = 26
VOCAB = 100000
EMB_DIM = 50
NUM_CONT = 13
D_IN = N_FIELDS * EMB_DIM + NUM_CONT
EPAD = 56

NW = 32
BPW = B // NW
CHUNK = 128
NCH = BPW // CHUNK
IPW = N_FIELDS * BPW

BT = 512
NT = B // BT


def _sc_gather(idx_flat, table2):
  mesh = plsc.VectorSubcoreMesh(core_axis_name="c", subcore_axis_name="s")

  @functools.partial(
      pl.kernel,
      mesh=mesh,
      compiler_params=pltpu.CompilerParams(use_tc_tiling_on_sc=False),
      out_type=jax.ShapeDtypeStruct((N_FIELDS, B, EPAD), jnp.float32),
      scratch_types=[
          pltpu.VMEM((IPW,), jnp.int32),
          pltpu.VMEM((CHUNK, EPAD), jnp.float32),
          pltpu.VMEM((CHUNK, EPAD), jnp.float32),
          pltpu.SemaphoreType.DMA,
          pltpu.SemaphoreType.DMA,
      ],
  )
  def gather_kernel(idx_hbm, tab_hbm, x_hbm, idx_v, gbuf0, gbuf1, sem0, sem1):
    w = lax.axis_index("s") * 2 + lax.axis_index("c")
    base = w * BPW
    pltpu.sync_copy(idx_hbm.at[pl.ds(w * IPW, IPW)], idx_v)

    @pl.loop(0, N_FIELDS)
    def _(f):
      @pl.loop(0, NCH)
      def _(j):
        q = f * NCH + j
        pltpu.async_copy(
            tab_hbm.at[idx_v.at[pl.ds(q * CHUNK, CHUNK)]], gbuf0, sem0).wait()
        pltpu.sync_copy(gbuf0, x_hbm.at[f, pl.ds(base + j * CHUNK, CHUNK)])

  return gather_kernel(idx_flat, table2)


def _mm_stats_kernel(x3_ref, c_ref, w_ref, wc_ref, b_ref, h_ref, s_ref):
  i = pl.program_id(0)
  xt = x3_ref[...].transpose(1, 0, 2).reshape(BT, N_FIELDS * EPAD)
  h = jnp.dot(xt, w_ref[...], preferred_element_type=jnp.float32)
  h = h + jnp.dot(c_ref[...], wc_ref[...], preferred_element_type=jnp.float32)
  h = h + b_ref[...]
  h_ref[...] = h
  ps = jnp.concatenate(
      [jnp.sum(h, 0, keepdims=True), jnp.sum(h * h, 0, keepdims=True)], 0)

  @pl.when(i == 0)
  def _():
    s_ref[...] = ps

  @pl.when(i > 0)
  def _():
    s_ref[...] = s_ref[...] + ps


def _bn_mm_stats_kernel(h_ref, s_in_ref, g_ref, be_ref, w_ref, b_ref,
                        h_out_ref, s_ref):
  i = pl.program_id(0)
  m = s_in_ref[0:1, :] * (1.0 / B)
  var = s_in_ref[1:2, :] * (1.0 / B) - m * m
  inv = lax.rsqrt(var + 1e-5)
  xn = (h_ref[...] - m) * (inv * g_ref[...]) + be_ref[...]
  xn = jnp.maximum(xn, 0.0)
  h = jnp.dot(xn, w_ref[...], preferred_element_type=jnp.float32) + b_ref[...]
  h_out_ref[...] = h
  ps = jnp.concatenate(
      [jnp.sum(h, 0, keepdims=True), jnp.sum(h * h, 0, keepdims=True)], 0)

  @pl.when(i == 0)
  def _():
    s_ref[...] = ps

  @pl.when(i > 0)
  def _():
    s_ref[...] = s_ref[...] + ps


def _final_kernel(h_ref, s_in_ref, g_ref, be_ref, w4_ref, b4_ref, w5_ref,
                  b5_ref, o_ref):
  m = s_in_ref[0:1, :] * (1.0 / B)
  var = s_in_ref[1:2, :] * (1.0 / B) - m * m
  inv = lax.rsqrt(var + 1e-5)
  xn = (h_ref[...] - m) * (inv * g_ref[...]) + be_ref[...]
  xn = jnp.maximum(xn, 0.0)
  u = jnp.dot(xn, w4_ref[...], preferred_element_type=jnp.float32) + b4_ref[...]
  u = jnp.maximum(u, 0.0)
  o_ref[...] = jnp.sum(u * w5_ref[...], axis=1, keepdims=True) + b5_ref[...]


def _const_spec(shape):
  return pl.BlockSpec(shape, lambda i: tuple(0 for _ in shape))


def kernel(cat_data, cont_data, tables, W1, b1, g1, be1, W2, b2, g2, be2,
           W3, b3, g3, be3, W4, b4, W5, b5):
  offs = (jnp.arange(N_FIELDS, dtype=jnp.int32) * VOCAB)[:, None]
  idx = cat_data.astype(jnp.int32).T + offs
  idx_flat = (idx.reshape(N_FIELDS, NW, BPW)
              .transpose(1, 0, 2).reshape(NW * IPW))
  table2 = tables.reshape(N_FIELDS * VOCAB, EMB_DIM)
  table_p = jnp.pad(table2, ((0, 0), (0, EPAD - EMB_DIM)))

  x3 = _sc_gather(idx_flat, table_p)

  d1, d2, d3, d4 = 512, 256, 128, 64
  w1a = W1[:, :N_FIELDS * EMB_DIM].T.reshape(N_FIELDS, EMB_DIM, d1)
  w1a = jnp.pad(w1a, ((0, 0), (0, EPAD - EMB_DIM), (0, 0)))
  w1a = w1a.reshape(N_FIELDS * EPAD, d1)
  w1c = W1[:, N_FIELDS * EMB_DIM:].T
  h1, s1 = pl.pallas_call(
      _mm_stats_kernel,
      grid=(NT,),
      in_specs=[
          pl.BlockSpec((N_FIELDS, BT, EPAD), lambda i: (0, i, 0)),
          pl.BlockSpec((BT, NUM_CONT), lambda i: (i, 0)),
          _const_spec((N_FIELDS * EPAD, d1)),
          _const_spec((NUM_CONT, d1)),
          _const_spec((1, d1)),
      ],
      out_specs=[
          pl.BlockSpec((BT, d1), lambda i: (i, 0)),
          _const_spec((2, d1)),
      ],
      out_shape=[
          jax.ShapeDtypeStruct((B, d1), jnp.float32),
          jax.ShapeDtypeStruct((2, d1), jnp.float32),
      ],
  )(x3, cont_data, w1a, w1c, b1.reshape(1, d1))

  h2, s2 = pl.pallas_call(
      _bn_mm_stats_kernel,
      grid=(NT,),
      in_specs=[
          pl.BlockSpec((BT, d1), lambda i: (i, 0)),
          _const_spec((2, d1)),
          _const_spec((1, d1)),
          _const_spec((1, d1)),
          _const_spec((d1, d2)),
          _const_spec((1, d2)),
      ],
      out_specs=[
          pl.BlockSpec((BT, d2), lambda i: (i, 0)),
          _const_spec((2, d2)),
      ],
      out_shape=[
          jax.ShapeDtypeStruct((B, d2), jnp.float32),
          jax.ShapeDtypeStruct((2, d2), jnp.float32),
      ],
  )(h1, s1, g1.reshape(1, d1), be1.reshape(1, d1), W2.T, b2.reshape(1, d2))

  h3, s3 = pl.pallas_call(
      _bn_mm_stats_kernel,
      grid=(NT,),
      in_specs=[
          pl.BlockSpec((BT, d2), lambda i: (i, 0)),
          _const_spec((2, d2)),
          _const_spec((1, d2)),
          _const_spec((1, d2)),
          _const_spec((d2, d3)),
          _const_spec((1, d3)),
      ],
      out_specs=[
          pl.BlockSpec((BT, d3), lambda i: (i, 0)),
          _const_spec((2, d3)),
      ],
      out_shape=[
          jax.ShapeDtypeStruct((B, d3), jnp.float32),
          jax.ShapeDtypeStruct((2, d3), jnp.float32),
      ],
  )(h2, s2, g2.reshape(1, d2), be2.reshape(1, d2), W3.T, b3.reshape(1, d3))

  out = pl.pallas_call(
      _final_kernel,
      grid=(NT,),
      in_specs=[
          pl.BlockSpec((BT, d3), lambda i: (i, 0)),
          _const_spec((2, d3)),
          _const_spec((1, d3)),
          _const_spec((1, d3)),
          _const_spec((d3, d4)),
          _const_spec((1, d4)),
          _const_spec((1, d4)),
          _const_spec((1, 1)),
      ],
      out_specs=pl.BlockSpec((BT, 1), lambda i: (i, 0)),
      out_shape=jax.ShapeDtypeStruct((B, 1), jnp.float32),
  )(h3, s3, g3.reshape(1, d3), be3.reshape(1, d3), W4.T, b4.reshape(1, d4),
    W5, b5.reshape(1, 1))

  return out.reshape(B)


# speedup vs baseline: 5.4827x; 1.4021x over previous
import functools

import jax
import jax.numpy as jnp
from jax import lax
from jax.experimental import pallas as pl
from jax.experimental.pallas import tpu as pltpu
from jax.experimental.pallas import tpu_sc as plsc

B = 16384
N_FIELDS = 26
VOCAB = 100000
EMB_DIM = 50
NUM_CONT = 13
D_IN = N_FIELDS * EMB_DIM + NUM_CONT
EPAD = 56
TPAD = 128

NW = 32
BPW = B // NW
CHUNK = 128
NCH = BPW // CHUNK
IPW = N_FIELDS * BPW

BT = 512
NT = B // BT


def _sc_gather(idx_flat, table2, nf):
  mesh = plsc.VectorSubcoreMesh(core_axis_name="c", subcore_axis_name="s")
  ipw = nf * BPW

  @functools.partial(
      pl.kernel,
      mesh=mesh,
      compiler_params=pltpu.CompilerParams(use_tc_tiling_on_sc=False),
      out_type=jax.ShapeDtypeStruct((nf, B, EPAD), jnp.float32),
      scratch_types=[
          pltpu.VMEM((ipw,), jnp.int32),
          pltpu.VMEM((CHUNK, TPAD), jnp.float32),
          pltpu.VMEM((CHUNK, TPAD), jnp.float32),
          pltpu.SemaphoreType.DMA,
          pltpu.SemaphoreType.DMA,
      ],
  )
  def gather_kernel(idx_hbm, tab_hbm, x_hbm, idx_v, gbuf0, gbuf1, sem0, sem1):
    w = lax.axis_index("s") * 2 + lax.axis_index("c")
    base = w * BPW
    pltpu.sync_copy(idx_hbm.at[pl.ds(w * ipw, ipw)], idx_v)

    @pl.loop(0, nf)
    def _(f):
      @pl.loop(0, NCH)
      def _(j):
        q = f * NCH + j
        pltpu.async_copy(
            tab_hbm.at[idx_v.at[pl.ds(q * CHUNK, CHUNK)]], gbuf0, sem0).wait()
        pltpu.sync_copy(gbuf0.at[:, pl.ds(0, EPAD)],
                        x_hbm.at[f, pl.ds(base + j * CHUNK, CHUNK)])

  return gather_kernel(idx_flat, table2)


def _mm_stats_kernel(xa_ref, xb_ref, c_ref, wa_ref, wb_ref, wc_ref, b_ref,
                     h_ref, s_ref):
  i = pl.program_id(0)
  nfh = N_FIELDS // 2
  xa = xa_ref[...].transpose(1, 0, 2).reshape(BT, nfh * EPAD)
  xb = xb_ref[...].transpose(1, 0, 2).reshape(BT, nfh * EPAD)
  h = jnp.dot(xa, wa_ref[...], preferred_element_type=jnp.float32)
  h = h + jnp.dot(xb, wb_ref[...], preferred_element_type=jnp.float32)
  h = h + jnp.dot(c_ref[...], wc_ref[...], preferred_element_type=jnp.float32)
  h = h + b_ref[...]
  h_ref[...] = h
  ps = jnp.concatenate(
      [jnp.sum(h, 0, keepdims=True), jnp.sum(h * h, 0, keepdims=True)], 0)

  @pl.when(i == 0)
  def _():
    s_ref[...] = ps

  @pl.when(i > 0)
  def _():
    s_ref[...] = s_ref[...] + ps


def _bn_mm_stats_kernel(h_ref, s_in_ref, g_ref, be_ref, w_ref, b_ref,
                        h_out_ref, s_ref):
  i = pl.program_id(0)
  m = s_in_ref[0:1, :] * (1.0 / B)
  var = s_in_ref[1:2, :] * (1.0 / B) - m * m
  inv = lax.rsqrt(var + 1e-5)
  xn = (h_ref[...] - m) * (inv * g_ref[...]) + be_ref[...]
  xn = jnp.maximum(xn, 0.0)
  h = jnp.dot(xn, w_ref[...], preferred_element_type=jnp.float32) + b_ref[...]
  h_out_ref[...] = h
  ps = jnp.concatenate(
      [jnp.sum(h, 0, keepdims=True), jnp.sum(h * h, 0, keepdims=True)], 0)

  @pl.when(i == 0)
  def _():
    s_ref[...] = ps

  @pl.when(i > 0)
  def _():
    s_ref[...] = s_ref[...] + ps


def _final_kernel(h_ref, s_in_ref, g_ref, be_ref, w4_ref, b4_ref, w5_ref,
                  b5_ref, o_ref):
  m = s_in_ref[0:1, :] * (1.0 / B)
  var = s_in_ref[1:2, :] * (1.0 / B) - m * m
  inv = lax.rsqrt(var + 1e-5)
  xn = (h_ref[...] - m) * (inv * g_ref[...]) + be_ref[...]
  xn = jnp.maximum(xn, 0.0)
  u = jnp.dot(xn, w4_ref[...], preferred_element_type=jnp.float32) + b4_ref[...]
  u = jnp.maximum(u, 0.0)
  o_ref[...] = jnp.sum(u * w5_ref[...], axis=1, keepdims=True) + b5_ref[...]


def _const_spec(shape):
  return pl.BlockSpec(shape, lambda i: tuple(0 for _ in shape))


def kernel(cat_data, cont_data, tables, W1, b1, g1, be1, W2, b2, g2, be2,
           W3, b3, g3, be3, W4, b4, W5, b5):
  nfh = N_FIELDS // 2
  offs = (jnp.arange(nfh, dtype=jnp.int32) * VOCAB)[:, None]
  idx = cat_data.astype(jnp.int32).T

  def _idx_half(ih):
    return ((idx[ih * nfh:(ih + 1) * nfh] + offs)
            .reshape(nfh, NW, BPW).transpose(1, 0, 2).reshape(NW * nfh * BPW))

  def _tab_half(ih):
    tp = jnp.pad(tables[ih * nfh:(ih + 1) * nfh],
                 ((0, 0), (0, 0), (0, TPAD - EMB_DIM)))
    return tp.reshape(nfh * VOCAB, TPAD)

  xa = _sc_gather(_idx_half(0), _tab_half(0), nfh)
  xb = _sc_gather(_idx_half(1), _tab_half(1), nfh)

  d1, d2, d3, d4 = 512, 256, 128, 64
  w1 = W1[:, :N_FIELDS * EMB_DIM].T.reshape(N_FIELDS, EMB_DIM, d1)
  w1 = jnp.pad(w1, ((0, 0), (0, EPAD - EMB_DIM), (0, 0)))
  w1a = w1[:nfh].reshape(nfh * EPAD, d1)
  w1b = w1[nfh:].reshape(nfh * EPAD, d1)
  w1c = W1[:, N_FIELDS * EMB_DIM:].T
  h1, s1 = pl.pallas_call(
      _mm_stats_kernel,
      grid=(NT,),
      in_specs=[
          pl.BlockSpec((nfh, BT, EPAD), lambda i: (0, i, 0)),
          pl.BlockSpec((nfh, BT, EPAD), lambda i: (0, i, 0)),
          pl.BlockSpec((BT, NUM_CONT), lambda i: (i, 0)),
          _const_spec((nfh * EPAD, d1)),
          _const_spec((nfh * EPAD, d1)),
          _const_spec((NUM_CONT, d1)),
          _const_spec((1, d1)),
      ],
      out_specs=[
          pl.BlockSpec((BT, d1), lambda i: (i, 0)),
          _const_spec((2, d1)),
      ],
      out_shape=[
          jax.ShapeDtypeStruct((B, d1), jnp.float32),
          jax.ShapeDtypeStruct((2, d1), jnp.float32),
      ],
  )(xa, xb, cont_data, w1a, w1b, w1c, b1.reshape(1, d1))

  h2, s2 = pl.pallas_call(
      _bn_mm_stats_kernel,
      grid=(NT,),
      in_specs=[
          pl.BlockSpec((BT, d1), lambda i: (i, 0)),
          _const_spec((2, d1)),
          _const_spec((1, d1)),
          _const_spec((1, d1)),
          _const_spec((d1, d2)),
          _const_spec((1, d2)),
      ],
      out_specs=[
          pl.BlockSpec((BT, d2), lambda i: (i, 0)),
          _const_spec((2, d2)),
      ],
      out_shape=[
          jax.ShapeDtypeStruct((B, d2), jnp.float32),
          jax.ShapeDtypeStruct((2, d2), jnp.float32),
      ],
  )(h1, s1, g1.reshape(1, d1), be1.reshape(1, d1), W2.T, b2.reshape(1, d2))

  h3, s3 = pl.pallas_call(
      _bn_mm_stats_kernel,
      grid=(NT,),
      in_specs=[
          pl.BlockSpec((BT, d2), lambda i: (i, 0)),
          _const_spec((2, d2)),
          _const_spec((1, d2)),
          _const_spec((1, d2)),
          _const_spec((d2, d3)),
          _const_spec((1, d3)),
      ],
      out_specs=[
          pl.BlockSpec((BT, d3), lambda i: (i, 0)),
          _const_spec((2, d3)),
      ],
      out_shape=[
          jax.ShapeDtypeStruct((B, d3), jnp.float32),
          jax.ShapeDtypeStruct((2, d3), jnp.float32),
      ],
  )(h2, s2, g2.reshape(1, d2), be2.reshape(1, d2), W3.T, b3.reshape(1, d3))

  out = pl.pallas_call(
      _final_kernel,
      grid=(NT,),
      in_specs=[
          pl.BlockSpec((BT, d3), lambda i: (i, 0)),
          _const_spec((2, d3)),
          _const_spec((1, d3)),
          _const_spec((1, d3)),
          _const_spec((d3, d4)),
          _const_spec((1, d4)),
          _const_spec((1, d4)),
          _const_spec((1, 1)),
      ],
      out_specs=pl.BlockSpec((BT, 1), lambda i: (i, 0)),
      out_shape=jax.ShapeDtypeStruct((B, 1), jnp.float32),
  )(h3, s3, g3.reshape(1, d3), be3.reshape(1, d3), W4.T, b4.reshape(1, d4),
    W5, b5.reshape(1, 1))

  return out.reshape(B)


# speedup vs baseline: 6.2500x; 1.1400x over previous
import functools

import jax
import jax.numpy as jnp
from jax import lax
from jax.experimental import pallas as pl
from jax.experimental.pallas import tpu as pltpu
from jax.experimental.pallas import tpu_sc as plsc

B = 16384
N_FIELDS = 26
VOCAB = 100000
EMB_DIM = 50
NUM_CONT = 13
D_IN = N_FIELDS * EMB_DIM + NUM_CONT
EPAD = 56
TPAD = 128

NW = 32
BPW = B // NW
CHUNK = 128
NCH = BPW // CHUNK
IPW = N_FIELDS * BPW

BT = 512
NT = B // BT


def _sc_gather(idx_flat, table2, nf):
  mesh = plsc.VectorSubcoreMesh(core_axis_name="c", subcore_axis_name="s")
  ipw = nf * BPW

  @functools.partial(
      pl.kernel,
      mesh=mesh,
      compiler_params=pltpu.CompilerParams(use_tc_tiling_on_sc=False),
      out_type=jax.ShapeDtypeStruct((nf, B, EPAD), jnp.float32),
      scratch_types=[
          pltpu.VMEM((ipw,), jnp.int32),
          pltpu.VMEM((NCH, CHUNK, TPAD), jnp.float32),
          pltpu.SemaphoreType.DMA,
      ],
  )
  def gather_kernel(idx_hbm, tab_hbm, x_hbm, idx_v, gbuf, sem):
    w = lax.axis_index("s") * 2 + lax.axis_index("c")
    base = w * BPW
    pltpu.sync_copy(idx_hbm.at[pl.ds(w * ipw, ipw)], idx_v)

    @pl.loop(0, nf)
    def _(f):
      hs = []
      for j in range(NCH):
        q = f * NCH + j
        hs.append(pltpu.async_copy(
            tab_hbm.at[idx_v.at[pl.ds(q * CHUNK, CHUNK)]], gbuf.at[j], sem))
      for j in range(NCH):
        hs[j].wait()
        pltpu.sync_copy(gbuf.at[j, :, pl.ds(0, EPAD)],
                        x_hbm.at[f, pl.ds(base + j * CHUNK, CHUNK)])

  return gather_kernel(idx_flat, table2)


def _mm_stats_kernel(x3_ref, c_ref, w_ref, wc_ref, b_ref, h_ref, s_ref):
  i = pl.program_id(0)
  xt = x3_ref[...].transpose(1, 0, 2).reshape(BT, N_FIELDS * EPAD)
  h = jnp.dot(xt, w_ref[...], preferred_element_type=jnp.float32)
  h = h + jnp.dot(c_ref[...], wc_ref[...], preferred_element_type=jnp.float32)
  h = h + b_ref[...]
  h_ref[...] = h
  ps = jnp.concatenate(
      [jnp.sum(h, 0, keepdims=True), jnp.sum(h * h, 0, keepdims=True)], 0)

  @pl.when(i == 0)
  def _():
    s_ref[...] = ps

  @pl.when(i > 0)
  def _():
    s_ref[...] = s_ref[...] + ps


def _bn_mm_stats_kernel(h_ref, s_in_ref, g_ref, be_ref, w_ref, b_ref,
                        h_out_ref, s_ref):
  i = pl.program_id(0)
  m = s_in_ref[0:1, :] * (1.0 / B)
  var = s_in_ref[1:2, :] * (1.0 / B) - m * m
  inv = lax.rsqrt(var + 1e-5)
  xn = (h_ref[...] - m) * (inv * g_ref[...]) + be_ref[...]
  xn = jnp.maximum(xn, 0.0)
  h = jnp.dot(xn, w_ref[...], preferred_element_type=jnp.float32) + b_ref[...]
  h_out_ref[...] = h
  ps = jnp.concatenate(
      [jnp.sum(h, 0, keepdims=True), jnp.sum(h * h, 0, keepdims=True)], 0)

  @pl.when(i == 0)
  def _():
    s_ref[...] = ps

  @pl.when(i > 0)
  def _():
    s_ref[...] = s_ref[...] + ps


def _final_kernel(h_ref, s_in_ref, g_ref, be_ref, w4_ref, b4_ref, w5_ref,
                  b5_ref, o_ref):
  m = s_in_ref[0:1, :] * (1.0 / B)
  var = s_in_ref[1:2, :] * (1.0 / B) - m * m
  inv = lax.rsqrt(var + 1e-5)
  xn = (h_ref[...] - m) * (inv * g_ref[...]) + be_ref[...]
  xn = jnp.maximum(xn, 0.0)
  u = jnp.dot(xn, w4_ref[...], preferred_element_type=jnp.float32) + b4_ref[...]
  u = jnp.maximum(u, 0.0)
  o_ref[...] = jnp.sum(u * w5_ref[...], axis=1, keepdims=True) + b5_ref[...]


def _const_spec(shape):
  return pl.BlockSpec(shape, lambda i: tuple(0 for _ in shape))


def kernel(cat_data, cont_data, tables, W1, b1, g1, be1, W2, b2, g2, be2,
           W3, b3, g3, be3, W4, b4, W5, b5):
  offs = (jnp.arange(N_FIELDS, dtype=jnp.int32) * VOCAB)[:, None]
  idx = cat_data.astype(jnp.int32).T + offs
  idx_flat = (idx.reshape(N_FIELDS, NW, BPW)
              .transpose(1, 0, 2).reshape(NW * IPW))
  table_p = jnp.pad(tables, ((0, 0), (0, 0), (0, TPAD - EMB_DIM)))
  table_p = table_p.reshape(N_FIELDS * VOCAB, TPAD)

  x3 = _sc_gather(idx_flat, table_p, N_FIELDS)

  d1, d2, d3, d4 = 512, 256, 128, 64
  w1a = W1[:, :N_FIELDS * EMB_DIM].T.reshape(N_FIELDS, EMB_DIM, d1)
  w1a = jnp.pad(w1a, ((0, 0), (0, EPAD - EMB_DIM), (0, 0)))
  w1a = w1a.reshape(N_FIELDS * EPAD, d1)
  w1c = W1[:, N_FIELDS * EMB_DIM:].T
  h1, s1 = pl.pallas_call(
      _mm_stats_kernel,
      grid=(NT,),
      in_specs=[
          pl.BlockSpec((N_FIELDS, BT, EPAD), lambda i: (0, i, 0)),
          pl.BlockSpec((BT, NUM_CONT), lambda i: (i, 0)),
          _const_spec((N_FIELDS * EPAD, d1)),
          _const_spec((NUM_CONT, d1)),
          _const_spec((1, d1)),
      ],
      out_specs=[
          pl.BlockSpec((BT, d1), lambda i: (i, 0)),
          _const_spec((2, d1)),
      ],
      out_shape=[
          jax.ShapeDtypeStruct((B, d1), jnp.float32),
          jax.ShapeDtypeStruct((2, d1), jnp.float32),
      ],
  )(x3, cont_data, w1a, w1c, b1.reshape(1, d1))

  h2, s2 = pl.pallas_call(
      _bn_mm_stats_kernel,
      grid=(NT,),
      in_specs=[
          pl.BlockSpec((BT, d1), lambda i: (i, 0)),
          _const_spec((2, d1)),
          _const_spec((1, d1)),
          _const_spec((1, d1)),
          _const_spec((d1, d2)),
          _const_spec((1, d2)),
      ],
      out_specs=[
          pl.BlockSpec((BT, d2), lambda i: (i, 0)),
          _const_spec((2, d2)),
      ],
      out_shape=[
          jax.ShapeDtypeStruct((B, d2), jnp.float32),
          jax.ShapeDtypeStruct((2, d2), jnp.float32),
      ],
  )(h1, s1, g1.reshape(1, d1), be1.reshape(1, d1), W2.T, b2.reshape(1, d2))

  h3, s3 = pl.pallas_call(
      _bn_mm_stats_kernel,
      grid=(NT,),
      in_specs=[
          pl.BlockSpec((BT, d2), lambda i: (i, 0)),
          _const_spec((2, d2)),
          _const_spec((1, d2)),
          _const_spec((1, d2)),
          _const_spec((d2, d3)),
          _const_spec((1, d3)),
      ],
      out_specs=[
          pl.BlockSpec((BT, d3), lambda i: (i, 0)),
          _const_spec((2, d3)),
      ],
      out_shape=[
          jax.ShapeDtypeStruct((B, d3), jnp.float32),
          jax.ShapeDtypeStruct((2, d3), jnp.float32),
      ],
  )(h2, s2, g2.reshape(1, d2), be2.reshape(1, d2), W3.T, b3.reshape(1, d3))

  out = pl.pallas_call(
      _final_kernel,
      grid=(NT,),
      in_specs=[
          pl.BlockSpec((BT, d3), lambda i: (i, 0)),
          _const_spec((2, d3)),
          _const_spec((1, d3)),
          _const_spec((1, d3)),
          _const_spec((d3, d4)),
          _const_spec((1, d4)),
          _const_spec((1, d4)),
          _const_spec((1, 1)),
      ],
      out_specs=pl.BlockSpec((BT, 1), lambda i: (i, 0)),
      out_shape=jax.ShapeDtypeStruct((B, 1), jnp.float32),
  )(h3, s3, g3.reshape(1, d3), be3.reshape(1, d3), W4.T, b4.reshape(1, d4),
    W5, b5.reshape(1, 1))

  return out.reshape(B)
